# baseline (speedup 1.0000x reference)
import dataclasses
import functools

import jax
import jax.numpy as jnp
from jax import lax
from jax.experimental import pallas as pl
from jax.experimental.pallas import tpu as pltpu
from jax.experimental.pallas import tpu_sc as plsc

N = 10000
E = 320000
D_IN = 128
D_HID = 256
D_OUT = 128

NC = 2
NS = 16
C = 128
N_CHUNKS = 160
EPT = N_CHUNKS * C
E_PAD = NS * EPT
N_CHUNKS_ES = N_CHUNKS // NC
EPT_ES = N_CHUNKS_ES * C
N_ACC = 10112
ZPT = N_ACC // NS
WPT = N_ACC // NS

_MESH = dict(core_axis_name="c", subcore_axis_name="s")

_SC_CP = pltpu.CompilerParams()
if "needs_layout_passes" in pltpu.CompilerParams.__dataclass_fields__:
    _SC_CP = dataclasses.replace(_SC_CP, needs_layout_passes=False)


def _ed3(srcg, dsts):
    s3 = srcg.reshape(NGL, GROUP, C)
    s3n = (srcg + N).reshape(NGL, GROUP, C)
    d3 = dsts.reshape(NGL, GROUP, C)
    return jnp.stack([s3, d3, s3n, d3], axis=1).reshape(4 * NGL, GROUP, C)


HPT = 80


def _deg(srcs, dsts):
    @functools.partial(
        pl.kernel,
        out_type=jax.ShapeDtypeStruct((NC * NS * 2 * HPT, 128), jnp.float32),
        mesh=plsc.VectorSubcoreMesh(**_MESH),
        compiler_params=_SC_CP,
        scratch_types=[
            pltpu.VMEM((HPT, 128), jnp.float32),
            pltpu.VMEM((HPT, 128), jnp.float32),
            pltpu.VMEM((2, C), jnp.int32),
            pltpu.SemaphoreType.DMA,
        ],
    )
    def k(srcs_hbm, dsts_hbm, out_hbm, acc_o, acc_i, idx2, dsem):
        c = lax.axis_index("c")
        s = lax.axis_index("s")
        w = c * NS + s

        @pl.loop(0, HPT)
        def _(r):
            @pl.loop(0, 128, step=16)
            def _(j):
                acc_o[r, pl.ds(j, 16)] = jnp.zeros((16,), jnp.float32)
                acc_i[r, pl.ds(j, 16)] = jnp.zeros((16,), jnp.float32)

        ones = jnp.full((16,), 1.0, jnp.float32)
        base_e = w * EPT_ES
        nce = N_CHUNKS_ES

        def count(e_hbm, acc):
            pltpu.async_copy(e_hbm.at[pl.ds(base_e, C)], idx2.at[0], dsem)

            @pl.loop(0, nce)
            def _(i):
                p = jnp.bitwise_and(i, 1)
                pltpu.make_async_copy(
                    e_hbm.at[pl.ds(base_e + i * C, C)], idx2.at[p],
                    dsem).wait()
                nxt = jnp.minimum(i + 1, nce - 1)
                pltpu.async_copy(
                    e_hbm.at[pl.ds(base_e + nxt * C, C)], idx2.at[1 - p],
                    dsem)
                for j in range(0, C, 16):
                    v = idx2[p, pl.ds(j, 16)]
                    plsc.addupdate_scatter(
                        acc, [jnp.right_shift(v, 7), jnp.bitwise_and(v, 127)],
                        ones)

            pltpu.make_async_copy(
                e_hbm.at[pl.ds(base_e + (nce - 1) * C, C)],
                idx2.at[(nce - 1) & 1 ^ 1], dsem).wait()

        count(srcs_hbm, acc_o)
        count(dsts_hbm, acc_i)

        pltpu.sync_copy(acc_o, out_hbm.at[pl.ds((w * 2) * HPT, HPT)])
        pltpu.sync_copy(acc_i, out_hbm.at[pl.ds((w * 2 + 1) * HPT, HPT)])

    return k(srcs, dsts)


def _deg_reduce(parts):
    def body(x_ref, o_ref):
        o_ref[...] = jnp.sum(x_ref[...], axis=0)

    return pl.pallas_call(
        body,
        in_specs=[pl.BlockSpec((NC * NS, 2, HPT, 128),
                               lambda: (0, 0, 0, 0))],
        out_specs=pl.BlockSpec((2, HPT, 128), lambda: (0, 0, 0)),
        out_shape=jax.ShapeDtypeStruct((2, HPT, 128), jnp.float32),
    )(parts)


GROUP = 4
NGL = E_PAD // (GROUP * C)
NG_FS = N_CHUNKS // GROUP
NG_ES = N_CHUNKS_ES // GROUP


def _make_agg(edge_split, tab_rows):
    NG = NG_ES if edge_split else NG_FS

    @functools.partial(
        pl.kernel,
        out_type=jax.ShapeDtypeStruct((NC * N_ACC, 128), jnp.float32),
        mesh=plsc.VectorSubcoreMesh(**_MESH),
        scratch_types=[
            pltpu.VMEM_SHARED((N_ACC, 128), jnp.float32),
            pltpu.VMEM((2, C, 128), jnp.float32),
            pltpu.VMEM((2, 2, GROUP, C), jnp.int32),
            pltpu.SemaphoreType.DMA((2,)),
            pltpu.SemaphoreType.DMA((2,)),
            pltpu.SemaphoreType.DMA,
        ],
    )
    def k(tab_hbm, ed3_hbm, zeros_hbm, out_hbm, acc, rows, ib, gsem,
          ssem, isem):
        c = lax.axis_index("c")
        s = lax.axis_index("s")
        if edge_split:
            gbase = (c * NS + s) * NG
            goff = 0
        else:
            gbase = s * NG
            goff = 2 * c
        g_last = gbase + NG - 1

        def idx_start(g, p):
            pltpu.async_copy(ed3_hbm.at[pl.ds(4 * g + goff, 2)], ib.at[p],
                             isem)

        def idx_wait(g, p):
            pltpu.make_async_copy(ed3_hbm.at[pl.ds(4 * g + goff, 2)],
                                  ib.at[p], isem).wait()

        def gather_start(p, j):
            pltpu.async_copy(tab_hbm.at[ib.at[p, 0, j]], rows.at[j & 1],
                             gsem.at[j & 1])

        def gather_wait(p, j):
            pltpu.make_async_copy(tab_hbm.at[ib.at[p, 0, j]], rows.at[j & 1],
                                  gsem.at[j & 1]).wait()

        def scat_start(p, j):
            pltpu.async_copy(rows.at[j & 1], acc.at[ib.at[p, 1, j]],
                             ssem.at[j & 1], add=True)

        def scat_wait(p, j):
            pltpu.make_async_copy(rows.at[j & 1], acc.at[ib.at[p, 1, j]],
                                  ssem.at[j & 1]).wait()

        idx_start(gbase, 0)
        pltpu.async_copy(zeros_hbm, acc.at[pl.ds(s * ZPT, ZPT)],
                         gsem.at[0])
        pltpu.make_async_copy(zeros_hbm, acc.at[pl.ds(s * ZPT, ZPT)],
                              gsem.at[0]).wait()
        plsc.subcore_barrier()

        def run_group(g, p, pn, first):
            idx_wait(g, p)
            for j in range(GROUP):
                if j < 2:
                    if not first:
                        scat_wait(pn, j + GROUP - 2)
                else:
                    scat_wait(p, j - 2)
                gather_start(p, j)
                if j < 1:
                    if not first:
                        gather_wait(pn, GROUP - 1)
                        scat_start(pn, GROUP - 1)
                else:
                    gather_wait(p, j - 1)
                    scat_start(p, j - 1)
                if j == 3:
                    idx_start(jnp.minimum(g + 1, g_last), pn)

        run_group(gbase, 0, 1, True)

        @pl.loop(1, NG)
        def _(gl):
            p = jnp.bitwise_and(gl, 1)
            run_group(gbase + gl, p, 1 - p, False)

        p_last = (NG - 1) & 1
        gather_wait(p_last, GROUP - 1)
        scat_start(p_last, GROUP - 1)
        scat_wait(p_last, GROUP - 2)
        scat_wait(p_last, GROUP - 1)
        idx_wait(g_last, 1 - p_last)

        plsc.subcore_barrier()
        wbase = s * WPT
        pltpu.sync_copy(acc.at[pl.ds(wbase, WPT)],
                        out_hbm.at[pl.ds(c * N_ACC + wbase, WPT)])

    return k


_agg_fs = _make_agg(False, 2 * N)
_agg_es = _make_agg(True, N)


_R = 1000


def _rs(ref):
    return lax.rsqrt(jnp.maximum(ref[...], 1.0))


def _blk(w):
    return pl.BlockSpec((_R, w), lambda i: (i, 0))


def _full(shape):
    return pl.BlockSpec(shape, lambda i: (0, 0))


_DOT = dict(preferred_element_type=jnp.float32)


def _s0(x, dego):
    def body(x_ref, dego_ref, u_ref):
        u_ref[...] = x_ref[...] * _rs(dego_ref)

    return pl.pallas_call(
        body, grid=(N // _R,),
        in_specs=[_blk(128), _blk(1)],
        out_specs=_blk(128),
        out_shape=jax.ShapeDtypeStruct((N, 128), jnp.float32),
    )(x, dego)


def _s1(p0, p1, degi, dego, W1, b1, W3):
    def body(p0_ref, p1_ref, degi_ref, dego_ref, w1_ref, b1_ref, w3_ref,
             o0_ref, o1_ref):
        a = p0_ref[...] + p1_ref[...]
        t = jnp.dot(a * _rs(degi_ref), w1_ref[...], **_DOT) + b1_ref[...]
        t = jnp.maximum(t, 0.0)
        y = jnp.dot(t * _rs(dego_ref), w3_ref[...], **_DOT)
        o0_ref[...] = y[:, :128]
        o1_ref[...] = y[:, 128:]

    return pl.pallas_call(
        body, grid=(N // _R,),
        in_specs=[_blk(128), _blk(128), _blk(1), _blk(1),
                  _full((D_IN, D_HID)), _full((1, D_HID)),
                  _full((D_HID, D_HID))],
        out_specs=[_blk(128), _blk(128)],
        out_shape=[jax.ShapeDtypeStruct((N, 128), jnp.float32)] * 2,
    )(p0, p1, degi, dego, W1, b1, W3)


def _s2(a0, a1, degi, dego, b, W):
    def body(a0_ref, a1_ref, degi_ref, dego_ref, b_ref, w_ref,
             o0_ref, o1_ref):
        a = jnp.concatenate([a0_ref[...], a1_ref[...]], axis=1)
        t = jnp.maximum(a * _rs(degi_ref) + b_ref[...], 0.0)
        y = jnp.dot(t * _rs(dego_ref), w_ref[...], **_DOT)
        o0_ref[...] = y[:, :128]
        o1_ref[...] = y[:, 128:]

    return pl.pallas_call(
        body, grid=(N // _R,),
        in_specs=[_blk(128), _blk(128), _blk(1), _blk(1),
                  _full((1, D_HID)), _full((D_HID, D_HID))],
        out_specs=[_blk(128), _blk(128)],
        out_shape=[jax.ShapeDtypeStruct((N, 128), jnp.float32)] * 2,
    )(a0, a1, degi, dego, b, W)


def _s3(a0, a1, degi, dego, b, W):
    def body(a0_ref, a1_ref, degi_ref, dego_ref, b_ref, w_ref, o_ref):
        a = jnp.concatenate([a0_ref[...], a1_ref[...]], axis=1)
        t = jnp.maximum(a * _rs(degi_ref) + b_ref[...], 0.0)
        o_ref[...] = jnp.dot(t * _rs(dego_ref), w_ref[...], **_DOT)

    return pl.pallas_call(
        body, grid=(N // _R,),
        in_specs=[_blk(128), _blk(128), _blk(1), _blk(1),
                  _full((1, D_HID)), _full((D_HID, D_OUT))],
        out_specs=_blk(128),
        out_shape=jax.ShapeDtypeStruct((N, D_OUT), jnp.float32),
    )(a0, a1, degi, dego, b, W)


def _s4(p0, p1, degi, b2):
    def body(p0_ref, p1_ref, degi_ref, b_ref, o_ref):
        a = p0_ref[...] + p1_ref[...]
        o_ref[...] = a * _rs(degi_ref) + b_ref[...]

    return pl.pallas_call(
        body, grid=(N // _R,),
        in_specs=[_blk(128), _blk(128), _blk(1), _full((1, D_OUT))],
        out_specs=_blk(128),
        out_shape=jax.ShapeDtypeStruct((N, D_OUT), jnp.float32),
    )(p0, p1, degi, b2)


def _halves(arr2):
    return arr2[:N], arr2[N_ACC:N_ACC + N]


def kernel(inputs, edge_index, W1, b1, W3, b3, W4, b4, W2, b2):
    src = edge_index[0]
    dst = edge_index[1]
    pad = E_PAD - E
    srcg = jnp.concatenate([src, jnp.zeros((pad,), jnp.int32)])
    srcs = jnp.concatenate([src, jnp.full((pad,), N, jnp.int32)])
    dsts = jnp.concatenate([dst, jnp.full((pad,), N, jnp.int32)])

    ed3 = _ed3(srcg, dsts)
    zrows = jnp.zeros((ZPT, 128), jnp.float32)
    parts = _deg(srcs, dsts).reshape(NC * NS, 2, HPT, 128)
    degs = _deg_reduce(parts)
    dego = degs[0].reshape(-1)[:N, None]
    degi = degs[1].reshape(-1)[:N, None]

    u = _s0(inputs, dego)
    p = _agg_es(u, ed3, zrows)
    h0, h1 = _s1(*_halves(p), degi, dego, W1, b1.reshape(1, -1), W3)
    a = _agg_fs(jnp.concatenate([h0, h1]), ed3, zrows)
    h0, h1 = _s2(*_halves(a), degi, dego, b3.reshape(1, -1), W4)
    a = _agg_fs(jnp.concatenate([h0, h1]), ed3, zrows)
    u = _s3(*_halves(a), degi, dego, b4.reshape(1, -1), W2)
    p = _agg_es(u, ed3, zrows)
    return _s4(*_halves(p), degi, b2.reshape(1, -1))


# speedup vs baseline: 1.1646x; 1.1646x over previous
import dataclasses
import functools

import jax
import jax.numpy as jnp
from jax import lax
from jax.experimental import pallas as pl
from jax.experimental.pallas import tpu as pltpu
from jax.experimental.pallas import tpu_sc as plsc

N = 10000
E = 320000
D_IN = 128
D_HID = 256
D_OUT = 128

NC = 2
NS = 16
C = 128
N_CHUNKS = 160
EPT = N_CHUNKS * C
E_PAD = NS * EPT
N_CHUNKS_ES = N_CHUNKS // NC
EPT_ES = N_CHUNKS_ES * C
N_ACC = 10112
ZPT = N_ACC // NS
WPT = N_ACC // NS

_MESH = dict(core_axis_name="c", subcore_axis_name="s")

_SC_CP = pltpu.CompilerParams()
if "needs_layout_passes" in pltpu.CompilerParams.__dataclass_fields__:
    _SC_CP = dataclasses.replace(_SC_CP, needs_layout_passes=False)


def _ed3(srcg, dsts):
    s3 = srcg.reshape(NGL, GROUP, C)
    s3n = (srcg + N).reshape(NGL, GROUP, C)
    d3 = dsts.reshape(NGL, GROUP, C)
    return jnp.stack([s3, d3, s3n, d3], axis=1).reshape(4 * NGL, GROUP, C)


HPT = 80


def _deg(srcs, dsts):
    @functools.partial(
        pl.kernel,
        out_type=jax.ShapeDtypeStruct((NC * NS * 2 * HPT, 128), jnp.float32),
        mesh=plsc.VectorSubcoreMesh(**_MESH),
        compiler_params=_SC_CP,
        scratch_types=[
            pltpu.VMEM((HPT, 128), jnp.float32),
            pltpu.VMEM((HPT, 128), jnp.float32),
            pltpu.VMEM((2, C), jnp.int32),
            pltpu.SemaphoreType.DMA,
        ],
    )
    def k(srcs_hbm, dsts_hbm, out_hbm, acc_o, acc_i, idx2, dsem):
        c = lax.axis_index("c")
        s = lax.axis_index("s")
        w = c * NS + s

        @pl.loop(0, HPT)
        def _(r):
            @pl.loop(0, 128, step=16)
            def _(j):
                acc_o[r, pl.ds(j, 16)] = jnp.zeros((16,), jnp.float32)
                acc_i[r, pl.ds(j, 16)] = jnp.zeros((16,), jnp.float32)

        ones = jnp.full((16,), 1.0, jnp.float32)
        base_e = w * EPT_ES
        nce = N_CHUNKS_ES

        def count(e_hbm, acc):
            pltpu.async_copy(e_hbm.at[pl.ds(base_e, C)], idx2.at[0], dsem)

            @pl.loop(0, nce)
            def _(i):
                p = jnp.bitwise_and(i, 1)
                pltpu.make_async_copy(
                    e_hbm.at[pl.ds(base_e + i * C, C)], idx2.at[p],
                    dsem).wait()
                nxt = jnp.minimum(i + 1, nce - 1)
                pltpu.async_copy(
                    e_hbm.at[pl.ds(base_e + nxt * C, C)], idx2.at[1 - p],
                    dsem)
                for j in range(0, C, 16):
                    v = idx2[p, pl.ds(j, 16)]
                    plsc.addupdate_scatter(
                        acc, [jnp.right_shift(v, 7), jnp.bitwise_and(v, 127)],
                        ones)

            pltpu.make_async_copy(
                e_hbm.at[pl.ds(base_e + (nce - 1) * C, C)],
                idx2.at[(nce - 1) & 1 ^ 1], dsem).wait()

        count(srcs_hbm, acc_o)
        count(dsts_hbm, acc_i)

        pltpu.sync_copy(acc_o, out_hbm.at[pl.ds((w * 2) * HPT, HPT)])
        pltpu.sync_copy(acc_i, out_hbm.at[pl.ds((w * 2 + 1) * HPT, HPT)])

    return k(srcs, dsts)


def _deg_reduce(parts):
    def body(x_ref, o_ref):
        o_ref[...] = jnp.sum(x_ref[...], axis=0)

    return pl.pallas_call(
        body,
        in_specs=[pl.BlockSpec((NC * NS, 2, HPT, 128),
                               lambda: (0, 0, 0, 0))],
        out_specs=pl.BlockSpec((2, HPT, 128), lambda: (0, 0, 0)),
        out_shape=jax.ShapeDtypeStruct((2, HPT, 128), jnp.float32),
    )(parts)


GROUP = 8
NGL = E_PAD // (GROUP * C)
NG_FS = N_CHUNKS // GROUP
NG_ES = N_CHUNKS_ES // GROUP


def _make_agg(edge_split, tab_rows):
    NG = NG_ES if edge_split else NG_FS

    @functools.partial(
        pl.kernel,
        out_type=jax.ShapeDtypeStruct((NC * N_ACC, 128), jnp.float32),
        mesh=plsc.VectorSubcoreMesh(**_MESH),
        scratch_types=[
            pltpu.VMEM_SHARED((N_ACC, 128), jnp.float32),
            pltpu.VMEM((2, C, 128), jnp.float32),
            pltpu.VMEM((2, 2, GROUP, C), jnp.int32),
            pltpu.SemaphoreType.DMA((2,)),
            pltpu.SemaphoreType.DMA((2,)),
            pltpu.SemaphoreType.DMA,
        ],
    )
    def k(tab_hbm, ed3_hbm, zeros_hbm, out_hbm, acc, rows, ib, gsem,
          ssem, isem):
        c = lax.axis_index("c")
        s = lax.axis_index("s")
        if edge_split:
            gbase = (c * NS + s) * NG
            goff = 0
        else:
            gbase = s * NG
            goff = 2 * c
        g_last = gbase + NG - 1

        def idx_start(g, p):
            pltpu.async_copy(ed3_hbm.at[pl.ds(4 * g + goff, 2)], ib.at[p],
                             isem)

        def idx_wait(g, p):
            pltpu.make_async_copy(ed3_hbm.at[pl.ds(4 * g + goff, 2)],
                                  ib.at[p], isem).wait()

        def gather_start(p, j):
            pltpu.async_copy(tab_hbm.at[ib.at[p, 0, j]], rows.at[j & 1],
                             gsem.at[j & 1])

        def gather_wait(p, j):
            pltpu.make_async_copy(tab_hbm.at[ib.at[p, 0, j]], rows.at[j & 1],
                                  gsem.at[j & 1]).wait()

        def scat_start(p, j):
            pltpu.async_copy(rows.at[j & 1], acc.at[ib.at[p, 1, j]],
                             ssem.at[j & 1], add=True)

        def scat_wait(p, j):
            pltpu.make_async_copy(rows.at[j & 1], acc.at[ib.at[p, 1, j]],
                                  ssem.at[j & 1]).wait()

        idx_start(gbase, 0)
        pltpu.async_copy(zeros_hbm, acc.at[pl.ds(s * ZPT, ZPT)],
                         gsem.at[0])
        pltpu.make_async_copy(zeros_hbm, acc.at[pl.ds(s * ZPT, ZPT)],
                              gsem.at[0]).wait()
        plsc.subcore_barrier()

        def run_group(g, p, pn, first):
            idx_wait(g, p)
            for j in range(GROUP):
                if j < 2:
                    if not first:
                        scat_wait(pn, j + GROUP - 2)
                else:
                    scat_wait(p, j - 2)
                gather_start(p, j)
                if j < 1:
                    if not first:
                        gather_wait(pn, GROUP - 1)
                        scat_start(pn, GROUP - 1)
                else:
                    gather_wait(p, j - 1)
                    scat_start(p, j - 1)
                if j == 3:
                    idx_start(jnp.minimum(g + 1, g_last), pn)

        run_group(gbase, 0, 1, True)

        @pl.loop(1, NG)
        def _(gl):
            p = jnp.bitwise_and(gl, 1)
            run_group(gbase + gl, p, 1 - p, False)

        p_last = (NG - 1) & 1
        gather_wait(p_last, GROUP - 1)
        scat_start(p_last, GROUP - 1)
        scat_wait(p_last, GROUP - 2)
        scat_wait(p_last, GROUP - 1)
        idx_wait(g_last, 1 - p_last)

        plsc.subcore_barrier()
        wbase = s * WPT
        pltpu.sync_copy(acc.at[pl.ds(wbase, WPT)],
                        out_hbm.at[pl.ds(c * N_ACC + wbase, WPT)])

    return k


_agg_fs = _make_agg(False, 2 * N)
_agg_es = _make_agg(True, N)


_R = 1000


def _rs(ref):
    return lax.rsqrt(jnp.maximum(ref[...], 1.0))


def _blk(w):
    return pl.BlockSpec((_R, w), lambda i: (i, 0))


def _full(shape):
    return pl.BlockSpec(shape, lambda i: (0, 0))


_DOT = dict(preferred_element_type=jnp.float32)


def _s0(x, dego):
    def body(x_ref, dego_ref, u_ref):
        u_ref[...] = x_ref[...] * _rs(dego_ref)

    return pl.pallas_call(
        body, grid=(N // _R,),
        in_specs=[_blk(128), _blk(1)],
        out_specs=_blk(128),
        out_shape=jax.ShapeDtypeStruct((N, 128), jnp.float32),
    )(x, dego)


def _s1(p0, p1, degi, dego, W1, b1, W3):
    def body(p0_ref, p1_ref, degi_ref, dego_ref, w1_ref, b1_ref, w3_ref,
             o0_ref, o1_ref):
        a = p0_ref[...] + p1_ref[...]
        t = jnp.dot(a * _rs(degi_ref), w1_ref[...], **_DOT) + b1_ref[...]
        t = jnp.maximum(t, 0.0)
        y = jnp.dot(t * _rs(dego_ref), w3_ref[...], **_DOT)
        o0_ref[...] = y[:, :128]
        o1_ref[...] = y[:, 128:]

    return pl.pallas_call(
        body, grid=(N // _R,),
        in_specs=[_blk(128), _blk(128), _blk(1), _blk(1),
                  _full((D_IN, D_HID)), _full((1, D_HID)),
                  _full((D_HID, D_HID))],
        out_specs=[_blk(128), _blk(128)],
        out_shape=[jax.ShapeDtypeStruct((N, 128), jnp.float32)] * 2,
    )(p0, p1, degi, dego, W1, b1, W3)


def _s2(a0, a1, degi, dego, b, W):
    def body(a0_ref, a1_ref, degi_ref, dego_ref, b_ref, w_ref,
             o0_ref, o1_ref):
        a = jnp.concatenate([a0_ref[...], a1_ref[...]], axis=1)
        t = jnp.maximum(a * _rs(degi_ref) + b_ref[...], 0.0)
        y = jnp.dot(t * _rs(dego_ref), w_ref[...], **_DOT)
        o0_ref[...] = y[:, :128]
        o1_ref[...] = y[:, 128:]

    return pl.pallas_call(
        body, grid=(N // _R,),
        in_specs=[_blk(128), _blk(128), _blk(1), _blk(1),
                  _full((1, D_HID)), _full((D_HID, D_HID))],
        out_specs=[_blk(128), _blk(128)],
        out_shape=[jax.ShapeDtypeStruct((N, 128), jnp.float32)] * 2,
    )(a0, a1, degi, dego, b, W)


def _s3(a0, a1, degi, dego, b, W):
    def body(a0_ref, a1_ref, degi_ref, dego_ref, b_ref, w_ref, o_ref):
        a = jnp.concatenate([a0_ref[...], a1_ref[...]], axis=1)
        t = jnp.maximum(a * _rs(degi_ref) + b_ref[...], 0.0)
        o_ref[...] = jnp.dot(t * _rs(dego_ref), w_ref[...], **_DOT)

    return pl.pallas_call(
        body, grid=(N // _R,),
        in_specs=[_blk(128), _blk(128), _blk(1), _blk(1),
                  _full((1, D_HID)), _full((D_HID, D_OUT))],
        out_specs=_blk(128),
        out_shape=jax.ShapeDtypeStruct((N, D_OUT), jnp.float32),
    )(a0, a1, degi, dego, b, W)


def _s4(p0, p1, degi, b2):
    def body(p0_ref, p1_ref, degi_ref, b_ref, o_ref):
        a = p0_ref[...] + p1_ref[...]
        o_ref[...] = a * _rs(degi_ref) + b_ref[...]

    return pl.pallas_call(
        body, grid=(N // _R,),
        in_specs=[_blk(128), _blk(128), _blk(1), _full((1, D_OUT))],
        out_specs=_blk(128),
        out_shape=jax.ShapeDtypeStruct((N, D_OUT), jnp.float32),
    )(p0, p1, degi, b2)


def _halves(arr2):
    return arr2[:N], arr2[N_ACC:N_ACC + N]


def kernel(inputs, edge_index, W1, b1, W3, b3, W4, b4, W2, b2):
    src = edge_index[0]
    dst = edge_index[1]
    pad = E_PAD - E
    srcg = jnp.concatenate([src, jnp.zeros((pad,), jnp.int32)])
    srcs = jnp.concatenate([src, jnp.full((pad,), N, jnp.int32)])
    dsts = jnp.concatenate([dst, jnp.full((pad,), N, jnp.int32)])

    ed3 = _ed3(srcg, dsts)
    zrows = jnp.zeros((ZPT, 128), jnp.float32)
    parts = _deg(srcs, dsts).reshape(NC * NS, 2, HPT, 128)
    degs = _deg_reduce(parts)
    dego = degs[0].reshape(-1)[:N, None]
    degi = degs[1].reshape(-1)[:N, None]

    u = _s0(inputs, dego)
    p = _agg_es(u, ed3, zrows)
    h0, h1 = _s1(*_halves(p), degi, dego, W1, b1.reshape(1, -1), W3)
    a = _agg_fs(jnp.concatenate([h0, h1]), ed3, zrows)
    h0, h1 = _s2(*_halves(a), degi, dego, b3.reshape(1, -1), W4)
    a = _agg_fs(jnp.concatenate([h0, h1]), ed3, zrows)
    u = _s3(*_halves(a), degi, dego, b4.reshape(1, -1), W2)
    p = _agg_es(u, ed3, zrows)
    return _s4(*_halves(p), degi, b2.reshape(1, -1))


# speedup vs baseline: 1.1691x; 1.0038x over previous
import dataclasses
import functools

import jax
import jax.numpy as jnp
from jax import lax
from jax.experimental import pallas as pl
from jax.experimental.pallas import tpu as pltpu
from jax.experimental.pallas import tpu_sc as plsc

N = 10000
E = 320000
D_IN = 128
D_HID = 256
D_OUT = 128

NC = 2
NS = 16
C = 128
N_CHUNKS = 160
EPT = N_CHUNKS * C
E_PAD = NS * EPT
N_CHUNKS_ES = N_CHUNKS // NC
EPT_ES = N_CHUNKS_ES * C
N_ACC = 10112
ZPT = N_ACC // NS
WPT = N_ACC // NS

_MESH = dict(core_axis_name="c", subcore_axis_name="s")

_SC_CP = pltpu.CompilerParams()
if "needs_layout_passes" in pltpu.CompilerParams.__dataclass_fields__:
    _SC_CP = dataclasses.replace(_SC_CP, needs_layout_passes=False)


def _ed3(srcg, dsts):
    s3 = srcg.reshape(NGL, GROUP, C)
    s3n = (srcg + N).reshape(NGL, GROUP, C)
    d3 = dsts.reshape(NGL, GROUP, C)
    return jnp.stack([s3, d3, s3n, d3], axis=1).reshape(4 * NGL, GROUP, C)


HPT = 80


def _deg(srcs, dsts):
    @functools.partial(
        pl.kernel,
        out_type=jax.ShapeDtypeStruct((NC * NS * 2 * HPT, 128), jnp.float32),
        mesh=plsc.VectorSubcoreMesh(**_MESH),
        compiler_params=_SC_CP,
        scratch_types=[
            pltpu.VMEM((HPT, 128), jnp.float32),
            pltpu.VMEM((HPT, 128), jnp.float32),
            pltpu.VMEM((2, C), jnp.int32),
            pltpu.SemaphoreType.DMA,
        ],
    )
    def k(srcs_hbm, dsts_hbm, out_hbm, acc_o, acc_i, idx2, dsem):
        c = lax.axis_index("c")
        s = lax.axis_index("s")
        w = c * NS + s

        @pl.loop(0, HPT)
        def _(r):
            @pl.loop(0, 128, step=16)
            def _(j):
                acc_o[r, pl.ds(j, 16)] = jnp.zeros((16,), jnp.float32)
                acc_i[r, pl.ds(j, 16)] = jnp.zeros((16,), jnp.float32)

        ones = jnp.full((16,), 1.0, jnp.float32)
        base_e = w * EPT_ES
        nce = N_CHUNKS_ES

        def count(e_hbm, acc):
            pltpu.async_copy(e_hbm.at[pl.ds(base_e, C)], idx2.at[0], dsem)

            @pl.loop(0, nce)
            def _(i):
                p = jnp.bitwise_and(i, 1)
                pltpu.make_async_copy(
                    e_hbm.at[pl.ds(base_e + i * C, C)], idx2.at[p],
                    dsem).wait()
                nxt = jnp.minimum(i + 1, nce - 1)
                pltpu.async_copy(
                    e_hbm.at[pl.ds(base_e + nxt * C, C)], idx2.at[1 - p],
                    dsem)
                for j in range(0, C, 16):
                    v = idx2[p, pl.ds(j, 16)]
                    plsc.addupdate_scatter(
                        acc, [jnp.right_shift(v, 7), jnp.bitwise_and(v, 127)],
                        ones)

            pltpu.make_async_copy(
                e_hbm.at[pl.ds(base_e + (nce - 1) * C, C)],
                idx2.at[(nce - 1) & 1 ^ 1], dsem).wait()

        count(srcs_hbm, acc_o)
        count(dsts_hbm, acc_i)

        pltpu.sync_copy(acc_o, out_hbm.at[pl.ds((w * 2) * HPT, HPT)])
        pltpu.sync_copy(acc_i, out_hbm.at[pl.ds((w * 2 + 1) * HPT, HPT)])

    return k(srcs, dsts)


def _deg_reduce(parts):
    def body(x_ref, o_ref):
        o_ref[...] = jnp.sum(x_ref[...], axis=0)

    return pl.pallas_call(
        body,
        in_specs=[pl.BlockSpec((NC * NS, 2, HPT, 128),
                               lambda: (0, 0, 0, 0))],
        out_specs=pl.BlockSpec((2, HPT, 128), lambda: (0, 0, 0)),
        out_shape=jax.ShapeDtypeStruct((2, HPT, 128), jnp.float32),
    )(parts)


GROUP = 8
NGL = E_PAD // (GROUP * C)
NG_FS = N_CHUNKS // GROUP
NG_ES = N_CHUNKS_ES // GROUP


def _make_agg(edge_split, tab_rows):
    NG = NG_ES if edge_split else NG_FS

    @functools.partial(
        pl.kernel,
        out_type=jax.ShapeDtypeStruct((NC * N_ACC, 128), jnp.float32),
        mesh=plsc.VectorSubcoreMesh(**_MESH),
        scratch_types=[
            pltpu.VMEM_SHARED((N_ACC, 128), jnp.float32),
            pltpu.VMEM((2, C, 128), jnp.float32),
            pltpu.VMEM((2, 2, GROUP, C), jnp.int32),
            pltpu.SemaphoreType.DMA((2,)),
            pltpu.SemaphoreType.DMA((2,)),
            pltpu.SemaphoreType.DMA,
        ],
    )
    def k(tab_hbm, ed3_hbm, zeros_hbm, out_hbm, acc, rows, ib, gsem,
          ssem, isem):
        c = lax.axis_index("c")
        s = lax.axis_index("s")
        if edge_split:
            gbase = (c * NS + s) * NG
            goff = 0
        else:
            gbase = s * NG
            goff = 2 * c
        g_last = gbase + NG - 1

        def idx_start(g, p):
            pltpu.async_copy(ed3_hbm.at[pl.ds(4 * g + goff, 2)], ib.at[p],
                             isem)

        def idx_wait(g, p):
            pltpu.make_async_copy(ed3_hbm.at[pl.ds(4 * g + goff, 2)],
                                  ib.at[p], isem).wait()

        def gather_start(p, j):
            pltpu.async_copy(tab_hbm.at[ib.at[p, 0, j]], rows.at[j & 1],
                             gsem.at[j & 1])

        def gather_wait(p, j):
            pltpu.make_async_copy(tab_hbm.at[ib.at[p, 0, j]], rows.at[j & 1],
                                  gsem.at[j & 1]).wait()

        def scat_start(p, j):
            pltpu.async_copy(rows.at[j & 1], acc.at[ib.at[p, 1, j]],
                             ssem.at[j & 1], add=True)

        def scat_wait(p, j):
            pltpu.make_async_copy(rows.at[j & 1], acc.at[ib.at[p, 1, j]],
                                  ssem.at[j & 1]).wait()

        idx_start(gbase, 0)
        pltpu.async_copy(zeros_hbm, acc.at[pl.ds(s * ZPT, ZPT)],
                         gsem.at[0])
        pltpu.make_async_copy(zeros_hbm, acc.at[pl.ds(s * ZPT, ZPT)],
                              gsem.at[0]).wait()
        plsc.subcore_barrier()

        def run_group(g, p, pn, first):
            idx_wait(g, p)
            for j in range(GROUP):
                if j < 2:
                    if not first:
                        scat_wait(pn, j + GROUP - 2)
                else:
                    scat_wait(p, j - 2)
                gather_start(p, j)
                if j < 1:
                    if not first:
                        gather_wait(pn, GROUP - 1)
                        scat_start(pn, GROUP - 1)
                else:
                    gather_wait(p, j - 1)
                    scat_start(p, j - 1)
                if j == 3:
                    idx_start(jnp.minimum(g + 1, g_last), pn)

        run_group(gbase, 0, 1, True)

        @pl.loop(1, NG)
        def _(gl):
            p = jnp.bitwise_and(gl, 1)
            run_group(gbase + gl, p, 1 - p, False)

        p_last = (NG - 1) & 1
        gather_wait(p_last, GROUP - 1)
        scat_start(p_last, GROUP - 1)
        scat_wait(p_last, GROUP - 2)
        scat_wait(p_last, GROUP - 1)
        idx_wait(g_last, 1 - p_last)

        plsc.subcore_barrier()
        wbase = s * WPT
        pltpu.sync_copy(acc.at[pl.ds(wbase, WPT)],
                        out_hbm.at[pl.ds(c * N_ACC + wbase, WPT)])

    return k


_agg_fs = _make_agg(False, 2 * N)
_agg_es = _make_agg(True, N)


_R = 2000


def _rs(ref):
    return lax.rsqrt(jnp.maximum(ref[...], 1.0))


def _blk(w):
    return pl.BlockSpec((_R, w), lambda i: (i, 0))


def _full(shape):
    return pl.BlockSpec(shape, lambda i: (0, 0))


_DOT = dict(preferred_element_type=jnp.float32)


def _s0(x, dego):
    def body(x_ref, dego_ref, u_ref):
        u_ref[...] = x_ref[...] * _rs(dego_ref)

    return pl.pallas_call(
        body, grid=(N // _R,),
        in_specs=[_blk(128), _blk(1)],
        out_specs=_blk(128),
        out_shape=jax.ShapeDtypeStruct((N, 128), jnp.float32),
    )(x, dego)


def _s1(p0, p1, degi, dego, W1, b1, W3):
    def body(p0_ref, p1_ref, degi_ref, dego_ref, w1_ref, b1_ref, w3_ref,
             o0_ref, o1_ref):
        a = p0_ref[...] + p1_ref[...]
        t = jnp.dot(a * _rs(degi_ref), w1_ref[...], **_DOT) + b1_ref[...]
        t = jnp.maximum(t, 0.0)
        y = jnp.dot(t * _rs(dego_ref), w3_ref[...], **_DOT)
        o0_ref[...] = y[:, :128]
        o1_ref[...] = y[:, 128:]

    return pl.pallas_call(
        body, grid=(N // _R,),
        in_specs=[_blk(128), _blk(128), _blk(1), _blk(1),
                  _full((D_IN, D_HID)), _full((1, D_HID)),
                  _full((D_HID, D_HID))],
        out_specs=[_blk(128), _blk(128)],
        out_shape=[jax.ShapeDtypeStruct((N, 128), jnp.float32)] * 2,
    )(p0, p1, degi, dego, W1, b1, W3)


def _s2(a0, a1, degi, dego, b, W):
    def body(a0_ref, a1_ref, degi_ref, dego_ref, b_ref, w_ref,
             o0_ref, o1_ref):
        a = jnp.concatenate([a0_ref[...], a1_ref[...]], axis=1)
        t = jnp.maximum(a * _rs(degi_ref) + b_ref[...], 0.0)
        y = jnp.dot(t * _rs(dego_ref), w_ref[...], **_DOT)
        o0_ref[...] = y[:, :128]
        o1_ref[...] = y[:, 128:]

    return pl.pallas_call(
        body, grid=(N // _R,),
        in_specs=[_blk(128), _blk(128), _blk(1), _blk(1),
                  _full((1, D_HID)), _full((D_HID, D_HID))],
        out_specs=[_blk(128), _blk(128)],
        out_shape=[jax.ShapeDtypeStruct((N, 128), jnp.float32)] * 2,
    )(a0, a1, degi, dego, b, W)


def _s3(a0, a1, degi, dego, b, W):
    def body(a0_ref, a1_ref, degi_ref, dego_ref, b_ref, w_ref, o_ref):
        a = jnp.concatenate([a0_ref[...], a1_ref[...]], axis=1)
        t = jnp.maximum(a * _rs(degi_ref) + b_ref[...], 0.0)
        o_ref[...] = jnp.dot(t * _rs(dego_ref), w_ref[...], **_DOT)

    return pl.pallas_call(
        body, grid=(N // _R,),
        in_specs=[_blk(128), _blk(128), _blk(1), _blk(1),
                  _full((1, D_HID)), _full((D_HID, D_OUT))],
        out_specs=_blk(128),
        out_shape=jax.ShapeDtypeStruct((N, D_OUT), jnp.float32),
    )(a0, a1, degi, dego, b, W)


def _s4(p0, p1, degi, b2):
    def body(p0_ref, p1_ref, degi_ref, b_ref, o_ref):
        a = p0_ref[...] + p1_ref[...]
        o_ref[...] = a * _rs(degi_ref) + b_ref[...]

    return pl.pallas_call(
        body, grid=(N // _R,),
        in_specs=[_blk(128), _blk(128), _blk(1), _full((1, D_OUT))],
        out_specs=_blk(128),
        out_shape=jax.ShapeDtypeStruct((N, D_OUT), jnp.float32),
    )(p0, p1, degi, b2)


def _halves(arr2):
    return arr2[:N], arr2[N_ACC:N_ACC + N]


def kernel(inputs, edge_index, W1, b1, W3, b3, W4, b4, W2, b2):
    src = edge_index[0]
    dst = edge_index[1]
    pad = E_PAD - E
    srcg = jnp.concatenate([src, jnp.zeros((pad,), jnp.int32)])
    srcs = jnp.concatenate([src, jnp.full((pad,), N, jnp.int32)])
    dsts = jnp.concatenate([dst, jnp.full((pad,), N, jnp.int32)])

    ed3 = _ed3(srcg, dsts)
    zrows = jnp.zeros((ZPT, 128), jnp.float32)
    parts = _deg(srcs, dsts).reshape(NC * NS, 2, HPT, 128)
    degs = _deg_reduce(parts)
    dego = degs[0].reshape(-1)[:N, None]
    degi = degs[1].reshape(-1)[:N, None]

    u = _s0(inputs, dego)
    p = _agg_es(u, ed3, zrows)
    h0, h1 = _s1(*_halves(p), degi, dego, W1, b1.reshape(1, -1), W3)
    a = _agg_fs(jnp.concatenate([h0, h1]), ed3, zrows)
    h0, h1 = _s2(*_halves(a), degi, dego, b3.reshape(1, -1), W4)
    a = _agg_fs(jnp.concatenate([h0, h1]), ed3, zrows)
    u = _s3(*_halves(a), degi, dego, b4.reshape(1, -1), W2)
    p = _agg_es(u, ed3, zrows)
    return _s4(*_halves(p), degi, b2.reshape(1, -1))


# speedup vs baseline: 1.1753x; 1.0054x over previous
import dataclasses
import functools

import jax
import jax.numpy as jnp
from jax import lax
from jax.experimental import pallas as pl
from jax.experimental.pallas import tpu as pltpu
from jax.experimental.pallas import tpu_sc as plsc

N = 10000
E = 320000
D_IN = 128
D_HID = 256
D_OUT = 128

NC = 2
NS = 16
C = 128
N_CHUNKS = 160
EPT = N_CHUNKS * C
E_PAD = NS * EPT
N_CHUNKS_ES = N_CHUNKS // NC
EPT_ES = N_CHUNKS_ES * C
N_ACC = 10112
ZPT = N_ACC // NS
WPT = N_ACC // NS

_MESH = dict(core_axis_name="c", subcore_axis_name="s")

_SC_CP = pltpu.CompilerParams()
if "needs_layout_passes" in pltpu.CompilerParams.__dataclass_fields__:
    _SC_CP = dataclasses.replace(_SC_CP, needs_layout_passes=False)


def _ed3(srcg, dsts):
    s3 = srcg.reshape(NGL, GROUP, C)
    s3n = (srcg + N).reshape(NGL, GROUP, C)
    d3 = dsts.reshape(NGL, GROUP, C)
    return jnp.stack([s3, d3, s3n, d3], axis=1).reshape(4 * NGL, GROUP, C)


HPT = 80


def _deg(srcs, dsts):
    @functools.partial(
        pl.kernel,
        out_type=jax.ShapeDtypeStruct((NC * NS * 2 * HPT, 128), jnp.float32),
        mesh=plsc.VectorSubcoreMesh(**_MESH),
        compiler_params=_SC_CP,
        scratch_types=[
            pltpu.VMEM((HPT, 128), jnp.float32),
            pltpu.VMEM((HPT, 128), jnp.float32),
            pltpu.VMEM((2, C), jnp.int32),
            pltpu.SemaphoreType.DMA,
        ],
    )
    def k(srcs_hbm, dsts_hbm, out_hbm, acc_o, acc_i, idx2, dsem):
        c = lax.axis_index("c")
        s = lax.axis_index("s")
        w = c * NS + s

        @pl.loop(0, HPT)
        def _(r):
            @pl.loop(0, 128, step=16)
            def _(j):
                acc_o[r, pl.ds(j, 16)] = jnp.zeros((16,), jnp.float32)
                acc_i[r, pl.ds(j, 16)] = jnp.zeros((16,), jnp.float32)

        ones = jnp.full((16,), 1.0, jnp.float32)
        base_e = w * EPT_ES
        nce = N_CHUNKS_ES

        def count(e_hbm, acc):
            pltpu.async_copy(e_hbm.at[pl.ds(base_e, C)], idx2.at[0], dsem)

            @pl.loop(0, nce)
            def _(i):
                p = jnp.bitwise_and(i, 1)
                pltpu.make_async_copy(
                    e_hbm.at[pl.ds(base_e + i * C, C)], idx2.at[p],
                    dsem).wait()
                nxt = jnp.minimum(i + 1, nce - 1)
                pltpu.async_copy(
                    e_hbm.at[pl.ds(base_e + nxt * C, C)], idx2.at[1 - p],
                    dsem)
                for j in range(0, C, 16):
                    v = idx2[p, pl.ds(j, 16)]
                    plsc.addupdate_scatter(
                        acc, [jnp.right_shift(v, 7), jnp.bitwise_and(v, 127)],
                        ones)

            pltpu.make_async_copy(
                e_hbm.at[pl.ds(base_e + (nce - 1) * C, C)],
                idx2.at[(nce - 1) & 1 ^ 1], dsem).wait()

        count(srcs_hbm, acc_o)
        count(dsts_hbm, acc_i)

        pltpu.sync_copy(acc_o, out_hbm.at[pl.ds((w * 2) * HPT, HPT)])
        pltpu.sync_copy(acc_i, out_hbm.at[pl.ds((w * 2 + 1) * HPT, HPT)])

    return k(srcs, dsts)


def _deg_reduce(parts):
    def body(x_ref, o_ref):
        o_ref[...] = jnp.sum(x_ref[...], axis=0)

    return pl.pallas_call(
        body,
        in_specs=[pl.BlockSpec((NC * NS, 2, HPT, 128),
                               lambda: (0, 0, 0, 0))],
        out_specs=pl.BlockSpec((2, HPT, 128), lambda: (0, 0, 0)),
        out_shape=jax.ShapeDtypeStruct((2, HPT, 128), jnp.float32),
    )(parts)


GROUP = 8
NGL = E_PAD // (GROUP * C)
NG_FS = N_CHUNKS // GROUP
NG_ES = N_CHUNKS_ES // GROUP


def _make_agg(edge_split, tab_rows):
    NG = NG_ES if edge_split else NG_FS

    @functools.partial(
        pl.kernel,
        out_type=jax.ShapeDtypeStruct((NC * N_ACC, 128), jnp.float32),
        mesh=plsc.VectorSubcoreMesh(**_MESH),
        scratch_types=[
            pltpu.VMEM_SHARED((N_ACC, 128), jnp.float32),
            pltpu.VMEM((2, C, 128), jnp.float32),
            pltpu.VMEM((2, 2, GROUP, C), jnp.int32),
            pltpu.SemaphoreType.DMA((2,)),
            pltpu.SemaphoreType.DMA((2,)),
            pltpu.SemaphoreType.DMA,
        ],
    )
    def k(tab_hbm, ed3_hbm, zeros_hbm, out_hbm, acc, rows, ib, gsem,
          ssem, isem):
        c = lax.axis_index("c")
        s = lax.axis_index("s")
        if edge_split:
            gbase = (c * NS + s) * NG
            goff = 0
        else:
            gbase = s * NG
            goff = 2 * c
        g_last = gbase + NG - 1

        def idx_start(g, p):
            pltpu.async_copy(ed3_hbm.at[pl.ds(4 * g + goff, 2)], ib.at[p],
                             isem)

        def idx_wait(g, p):
            pltpu.make_async_copy(ed3_hbm.at[pl.ds(4 * g + goff, 2)],
                                  ib.at[p], isem).wait()

        def gather_start(p, j):
            pltpu.async_copy(tab_hbm.at[ib.at[p, 0, j]], rows.at[j & 1],
                             gsem.at[j & 1])

        def gather_wait(p, j):
            pltpu.make_async_copy(tab_hbm.at[ib.at[p, 0, j]], rows.at[j & 1],
                                  gsem.at[j & 1]).wait()

        def scat_start(p, j):
            pltpu.async_copy(rows.at[j & 1], acc.at[ib.at[p, 1, j]],
                             ssem.at[j & 1], add=True)

        def scat_wait(p, j):
            pltpu.make_async_copy(rows.at[j & 1], acc.at[ib.at[p, 1, j]],
                                  ssem.at[j & 1]).wait()

        idx_start(gbase, 0)
        pltpu.async_copy(zeros_hbm, acc.at[pl.ds(s * ZPT, ZPT)],
                         gsem.at[0])
        pltpu.make_async_copy(zeros_hbm, acc.at[pl.ds(s * ZPT, ZPT)],
                              gsem.at[0]).wait()
        plsc.subcore_barrier()

        def run_group(g, p, pn, first):
            idx_wait(g, p)
            for j in range(GROUP):
                if j < 2:
                    if not first:
                        scat_wait(pn, j + GROUP - 2)
                else:
                    scat_wait(p, j - 2)
                gather_start(p, j)
                if j < 1:
                    if not first:
                        gather_wait(pn, GROUP - 1)
                        scat_start(pn, GROUP - 1)
                else:
                    gather_wait(p, j - 1)
                    scat_start(p, j - 1)
                if j == 3:
                    idx_start(jnp.minimum(g + 1, g_last), pn)

        run_group(gbase, 0, 1, True)

        @pl.loop(1, NG)
        def _(gl):
            p = jnp.bitwise_and(gl, 1)
            run_group(gbase + gl, p, 1 - p, False)

        p_last = (NG - 1) & 1
        gather_wait(p_last, GROUP - 1)
        scat_start(p_last, GROUP - 1)
        scat_wait(p_last, GROUP - 2)
        scat_wait(p_last, GROUP - 1)
        idx_wait(g_last, 1 - p_last)

        plsc.subcore_barrier()
        wbase = s * WPT
        pltpu.sync_copy(acc.at[pl.ds(wbase, WPT)],
                        out_hbm.at[pl.ds(c * N_ACC + wbase, WPT)])

    return k


_agg_fs = _make_agg(False, 2 * N)
_agg_es = _make_agg(True, N)


_R = 5000


def _rs(ref):
    return lax.rsqrt(jnp.maximum(ref[...], 1.0))


def _blk(w):
    return pl.BlockSpec((_R, w), lambda i: (i, 0))


def _full(shape):
    return pl.BlockSpec(shape, lambda i: (0, 0))


_DOT = dict(preferred_element_type=jnp.float32)


def _s0(x, dego):
    def body(x_ref, dego_ref, u_ref):
        u_ref[...] = x_ref[...] * _rs(dego_ref)

    return pl.pallas_call(
        body, grid=(N // _R,),
        in_specs=[_blk(128), _blk(1)],
        out_specs=_blk(128),
        out_shape=jax.ShapeDtypeStruct((N, 128), jnp.float32),
    )(x, dego)


def _s1(p0, p1, degi, dego, W1, b1, W3):
    def body(p0_ref, p1_ref, degi_ref, dego_ref, w1_ref, b1_ref, w3_ref,
             o0_ref, o1_ref):
        a = p0_ref[...] + p1_ref[...]
        t = jnp.dot(a * _rs(degi_ref), w1_ref[...], **_DOT) + b1_ref[...]
        t = jnp.maximum(t, 0.0)
        y = jnp.dot(t * _rs(dego_ref), w3_ref[...], **_DOT)
        o0_ref[...] = y[:, :128]
        o1_ref[...] = y[:, 128:]

    return pl.pallas_call(
        body, grid=(N // _R,),
        in_specs=[_blk(128), _blk(128), _blk(1), _blk(1),
                  _full((D_IN, D_HID)), _full((1, D_HID)),
                  _full((D_HID, D_HID))],
        out_specs=[_blk(128), _blk(128)],
        out_shape=[jax.ShapeDtypeStruct((N, 128), jnp.float32)] * 2,
    )(p0, p1, degi, dego, W1, b1, W3)


def _s2(a0, a1, degi, dego, b, W):
    def body(a0_ref, a1_ref, degi_ref, dego_ref, b_ref, w_ref,
             o0_ref, o1_ref):
        a = jnp.concatenate([a0_ref[...], a1_ref[...]], axis=1)
        t = jnp.maximum(a * _rs(degi_ref) + b_ref[...], 0.0)
        y = jnp.dot(t * _rs(dego_ref), w_ref[...], **_DOT)
        o0_ref[...] = y[:, :128]
        o1_ref[...] = y[:, 128:]

    return pl.pallas_call(
        body, grid=(N // _R,),
        in_specs=[_blk(128), _blk(128), _blk(1), _blk(1),
                  _full((1, D_HID)), _full((D_HID, D_HID))],
        out_specs=[_blk(128), _blk(128)],
        out_shape=[jax.ShapeDtypeStruct((N, 128), jnp.float32)] * 2,
    )(a0, a1, degi, dego, b, W)


def _s3(a0, a1, degi, dego, b, W):
    def body(a0_ref, a1_ref, degi_ref, dego_ref, b_ref, w_ref, o_ref):
        a = jnp.concatenate([a0_ref[...], a1_ref[...]], axis=1)
        t = jnp.maximum(a * _rs(degi_ref) + b_ref[...], 0.0)
        o_ref[...] = jnp.dot(t * _rs(dego_ref), w_ref[...], **_DOT)

    return pl.pallas_call(
        body, grid=(N // _R,),
        in_specs=[_blk(128), _blk(128), _blk(1), _blk(1),
                  _full((1, D_HID)), _full((D_HID, D_OUT))],
        out_specs=_blk(128),
        out_shape=jax.ShapeDtypeStruct((N, D_OUT), jnp.float32),
    )(a0, a1, degi, dego, b, W)


def _s4(p0, p1, degi, b2):
    def body(p0_ref, p1_ref, degi_ref, b_ref, o_ref):
        a = p0_ref[...] + p1_ref[...]
        o_ref[...] = a * _rs(degi_ref) + b_ref[...]

    return pl.pallas_call(
        body, grid=(N // _R,),
        in_specs=[_blk(128), _blk(128), _blk(1), _full((1, D_OUT))],
        out_specs=_blk(128),
        out_shape=jax.ShapeDtypeStruct((N, D_OUT), jnp.float32),
    )(p0, p1, degi, b2)


def _halves(arr2):
    return arr2[:N], arr2[N_ACC:N_ACC + N]


def kernel(inputs, edge_index, W1, b1, W3, b3, W4, b4, W2, b2):
    src = edge_index[0]
    dst = edge_index[1]
    pad = E_PAD - E
    srcg = jnp.concatenate([src, jnp.zeros((pad,), jnp.int32)])
    srcs = jnp.concatenate([src, jnp.full((pad,), N, jnp.int32)])
    dsts = jnp.concatenate([dst, jnp.full((pad,), N, jnp.int32)])

    ed3 = _ed3(srcg, dsts)
    zrows = jnp.zeros((ZPT, 128), jnp.float32)
    parts = _deg(srcs, dsts).reshape(NC * NS, 2, HPT, 128)
    degs = _deg_reduce(parts)
    dego = degs[0].reshape(-1)[:N, None]
    degi = degs[1].reshape(-1)[:N, None]

    u = _s0(inputs, dego)
    p = _agg_es(u, ed3, zrows)
    h0, h1 = _s1(*_halves(p), degi, dego, W1, b1.reshape(1, -1), W3)
    a = _agg_fs(jnp.concatenate([h0, h1]), ed3, zrows)
    h0, h1 = _s2(*_halves(a), degi, dego, b3.reshape(1, -1), W4)
    a = _agg_fs(jnp.concatenate([h0, h1]), ed3, zrows)
    u = _s3(*_halves(a), degi, dego, b4.reshape(1, -1), W2)
    p = _agg_es(u, ed3, zrows)
    return _s4(*_halves(p), degi, b2.reshape(1, -1))
